# Initial kernel scaffold; baseline (speedup 1.0000x reference)
import jax
import jax.numpy as jnp
from jax.experimental import pallas as pl


def kernel(x, edge_index, Wx0, Wh0, b0, Wx1, Wh1, b1, Wfc, bfc):
    raise NotImplementedError("write your pallas kernel here")


# speedup vs baseline: 2.8046x; 2.8046x over previous
import functools

import jax
import jax.numpy as jnp
from jax import lax
from jax.experimental import pallas as pl
from jax.experimental.pallas import tpu as pltpu
from jax.experimental.pallas import tpu_sc as plsc

NN = 10000
NP = 10240
EE = 160000
TT = 8
DD = 256
HH = 256

NC = 2
NS = 16
BB = 128
NB = 80
EP = NS * NB * BB
ZR = NP // NS


@functools.lru_cache(maxsize=None)
def _sc_kernels():
    mesh = plsc.VectorSubcoreMesh(core_axis_name="c", subcore_axis_name="s",
                                  num_cores=NC, num_subcores=NS)

    @functools.partial(
        pl.kernel,
        out_type=jax.ShapeDtypeStruct((2 * NP, 128), jnp.float32),
        mesh=mesh,
        scratch_types=[
            pltpu.VMEM((NB, BB), jnp.int32),
            pltpu.VMEM((NB, BB), jnp.int32),
            pltpu.VMEM((BB, 128), jnp.float32),
            pltpu.VMEM_SHARED((NP, 128), jnp.float32),
            pltpu.SemaphoreType.DMA,
        ],
    )
    def sc_agg(table_hbm, srcs_hbm, dsts_hbm, zeros_hbm, out_hbm,
               src_v, dst_v, rows_v, acc, sem):
        c = lax.axis_index("c")
        s = lax.axis_index("s")
        pltpu.sync_copy(srcs_hbm.at[c, s], src_v)
        pltpu.sync_copy(dsts_hbm.at[s], dst_v)
        pltpu.sync_copy(zeros_hbm, acc.at[pl.ds(s * ZR, ZR)])
        plsc.subcore_barrier()

        def body(nb, carry):
            pltpu.async_copy(table_hbm.at[src_v.at[nb]], rows_v, sem).wait()
            pltpu.sync_copy(rows_v, acc.at[dst_v.at[nb]], add=True)
            return carry

        lax.fori_loop(0, NB, body, 0)
        plsc.subcore_barrier()
        pltpu.sync_copy(acc.at[pl.ds(s * ZR, ZR)],
                        out_hbm.at[pl.ds(c * NP + s * ZR, ZR)])

    return sc_agg


def _xprep_body(x_ref, dinv_ref, out_ref):
    xs = x_ref[0] * dinv_ref[...]
    out_ref[0, 0] = xs[:, :128]
    out_ref[0, 1] = xs[:, 128:]


def _cell_body(accx_ref, acch_ref, dinv_ref, c_ref, wx_ref, wh_ref, b_ref,
               h_ref, cn_ref, hp_ref):
    d = dinv_ref[...]
    ax = jnp.concatenate([accx_ref[0], accx_ref[1]], axis=1) * d
    ah = jnp.concatenate([acch_ref[0], acch_ref[1]], axis=1) * d
    gates = (jnp.dot(ax, wx_ref[...], preferred_element_type=jnp.float32)
             + jnp.dot(ah, wh_ref[...], preferred_element_type=jnp.float32)
             + b_ref[...])
    i = jax.nn.sigmoid(gates[:, 0 * HH:1 * HH])
    f = jax.nn.sigmoid(gates[:, 1 * HH:2 * HH])
    g = jnp.tanh(gates[:, 2 * HH:3 * HH])
    o = jax.nn.sigmoid(gates[:, 3 * HH:4 * HH])
    cn = f * c_ref[...] + i * g
    h = o * jnp.tanh(cn)
    h_ref[...] = h
    cn_ref[...] = cn
    hp = h * d
    hp_ref[0, 0] = hp[:, :128]
    hp_ref[0, 1] = hp[:, 128:]


def _fc_body(h_ref, w_ref, b_ref, o_ref):
    o_ref[...] = jax.nn.sigmoid(
        jnp.dot(h_ref[...], w_ref[...], preferred_element_type=jnp.float32)
        + b_ref[...])


_BM = 256

_cell_call = pl.pallas_call(
    _cell_body,
    grid=(NP // _BM,),
    in_specs=[
        pl.BlockSpec((2, _BM, 128), lambda n: (0, n, 0)),
        pl.BlockSpec((2, _BM, 128), lambda n: (0, n, 0)),
        pl.BlockSpec((_BM, 1), lambda n: (n, 0)),
        pl.BlockSpec((_BM, HH), lambda n: (n, 0)),
        pl.BlockSpec((DD, 4 * HH), lambda n: (0, 0)),
        pl.BlockSpec((HH, 4 * HH), lambda n: (0, 0)),
        pl.BlockSpec((1, 4 * HH), lambda n: (0, 0)),
    ],
    out_specs=[
        pl.BlockSpec((_BM, HH), lambda n: (n, 0)),
        pl.BlockSpec((_BM, HH), lambda n: (n, 0)),
        pl.BlockSpec((1, 2, _BM, 128), lambda n: (0, 0, n, 0)),
    ],
    out_shape=[
        jax.ShapeDtypeStruct((NP, HH), jnp.float32),
        jax.ShapeDtypeStruct((NP, HH), jnp.float32),
        jax.ShapeDtypeStruct((1, 2, NP, 128), jnp.float32),
    ],
)

_xprep_call = pl.pallas_call(
    _xprep_body,
    grid=(TT, NP // _BM),
    in_specs=[
        pl.BlockSpec((1, _BM, DD), lambda t, n: (t, n, 0)),
        pl.BlockSpec((_BM, 1), lambda t, n: (n, 0)),
    ],
    out_specs=pl.BlockSpec((1, 2, _BM, 128), lambda t, n: (t, 0, n, 0)),
    out_shape=jax.ShapeDtypeStruct((TT, 2, NP, 128), jnp.float32),
)

_fc_call = pl.pallas_call(
    _fc_body,
    out_shape=jax.ShapeDtypeStruct((NP, 128), jnp.float32),
)


def kernel(x, edge_index, Wx0, Wh0, b0, Wx1, Wh1, b1, Wfc, bfc):
    src = edge_index[0].astype(jnp.int32)
    dst = edge_index[1].astype(jnp.int32)

    pad = EP - EE
    src_p = jnp.concatenate([src, jnp.full((pad,), NN, jnp.int32)])
    dst_p = jnp.concatenate([dst, jnp.full((pad,), NN, jnp.int32)])
    srcs = jnp.stack([src_p, src_p + NP]).reshape(NC, NS, NB, BB)
    dsts = dst_p.reshape(NS, NB, BB)

    zeros_agg = jnp.zeros((ZR, 128), jnp.float32)
    ones_tbl = jnp.ones((2 * NP, 128), jnp.float32)

    sc_agg_f = _sc_kernels()
    degp = sc_agg_f(ones_tbl, srcs, dsts, zeros_agg)
    deg = degp[:NP, 0]
    dinv = jax.lax.rsqrt(jnp.clip(deg, 1.0, None)).reshape(NP, 1)

    xpad = jnp.pad(x, ((0, 0), (0, NP - NN), (0, 0)))
    xp = _xprep_call(xpad, dinv).reshape(TT, 2 * NP, 128)

    agg = lambda tbl: sc_agg_f(tbl, srcs, dsts, zeros_agg)

    ax = [agg(xp[t]) for t in range(TT)]

    z2 = jnp.zeros((2, NP, 128), jnp.float32)
    zN = jnp.zeros((NP, HH), jnp.float32)
    b0r = b0.reshape(1, 4 * HH)
    b1r = b1.reshape(1, 4 * HH)

    g0 = z2
    g1 = z2
    c0 = zN
    c1 = zN
    h1 = zN
    for t in range(TT):
        axt = ax[t].reshape(2, NP, 128)
        _, c0, hp0 = _cell_call(axt, g0, dinv, c0, Wx0, Wh0, b0r)
        g0f = agg(hp0.reshape(2 * NP, 128))
        g0 = g0f.reshape(2, NP, 128)
        h1, c1, hp1 = _cell_call(g0, g1, dinv, c1, Wx1, Wh1, b1r)
        if t < TT - 1:
            g1 = agg(hp1.reshape(2 * NP, 128)).reshape(2, NP, 128)

    Wfc_pad = jnp.pad(Wfc, ((0, 0), (0, 127)))
    bfc_pad = jnp.pad(bfc, ((0, 127))).reshape(1, 128)
    score = _fc_call(h1, Wfc_pad, bfc_pad)
    return score[:NN, :1]


# speedup vs baseline: 3.2780x; 1.1688x over previous
import functools

import jax
import jax.numpy as jnp
from jax import lax
from jax.experimental import pallas as pl
from jax.experimental.pallas import tpu as pltpu
from jax.experimental.pallas import tpu_sc as plsc

NN = 10000
NP = 10240
EE = 160000
TT = 8
DD = 256
HH = 256

NC = 2
NS = 16
BB = 128
NB = 80
EP = NS * NB * BB
ZR = NP // NS


@functools.lru_cache(maxsize=None)
def _sc_kernels():
    mesh = plsc.VectorSubcoreMesh(core_axis_name="c", subcore_axis_name="s",
                                  num_cores=NC, num_subcores=NS)

    nbuf = 2
    CH = 16
    NCHUNK = NB // CH

    @functools.partial(
        pl.kernel,
        out_type=jax.ShapeDtypeStruct((2 * NP, 128), jnp.float32),
        mesh=mesh,
        scratch_types=[
            pltpu.VMEM((CH, BB), jnp.int32),
            pltpu.VMEM((CH, BB), jnp.int32),
            [pltpu.VMEM((BB, 128), jnp.float32)] * nbuf,
            pltpu.VMEM_SHARED((NP, 128), jnp.float32),
            [pltpu.SemaphoreType.DMA] * nbuf,
        ],
    )
    def sc_agg(table_hbm, srcs_hbm, dsts_hbm, zeros_hbm, out_hbm,
               src_v, dst_v, rows_v, acc, sems):
        c = lax.axis_index("c")
        s = lax.axis_index("s")
        pltpu.sync_copy(srcs_hbm.at[c, s, pl.ds(0, CH)], src_v)
        pltpu.sync_copy(dsts_hbm.at[s, pl.ds(0, CH)], dst_v)
        for b in range(nbuf):
            pltpu.async_copy(table_hbm.at[src_v.at[b]], rows_v[b], sems[b])
        pltpu.sync_copy(zeros_hbm, acc.at[pl.ds(s * ZR, ZR)])
        plsc.subcore_barrier()

        def chunk(k, carry):
            def inner(j, carry2):
                b = j % nbuf
                pltpu.make_async_copy(table_hbm.at[src_v.at[j]],
                                      rows_v[b], sems[b]).wait()
                pltpu.sync_copy(rows_v[b], acc.at[dst_v.at[j]], add=True)

                @pl.when(j + nbuf < CH)
                def _():
                    pltpu.async_copy(table_hbm.at[src_v.at[j + nbuf]],
                                     rows_v[b], sems[b])
                return carry2

            for j0 in range(0, CH - nbuf, nbuf):
                inner(j0, 0)
                inner(j0 + 1, 0)
            for j in range(CH - nbuf, CH):
                b = j % nbuf
                pltpu.make_async_copy(table_hbm.at[src_v.at[j]],
                                      rows_v[b], sems[b]).wait()
                pltpu.sync_copy(rows_v[b], acc.at[dst_v.at[j]], add=True)

            @pl.when(k + 1 < NCHUNK)
            def _():
                pltpu.sync_copy(srcs_hbm.at[c, s, pl.ds((k + 1) * CH, CH)],
                                src_v)
                pltpu.sync_copy(dsts_hbm.at[s, pl.ds((k + 1) * CH, CH)],
                                dst_v)
                for b in range(nbuf):
                    pltpu.async_copy(table_hbm.at[src_v.at[b]],
                                     rows_v[b], sems[b])
            return carry

        lax.fori_loop(0, NCHUNK, chunk, 0)
        plsc.subcore_barrier()
        pltpu.sync_copy(acc.at[pl.ds(s * ZR, ZR)],
                        out_hbm.at[pl.ds(c * NP + s * ZR, ZR)])

    return sc_agg


def _xprep_body(x_ref, dinv_ref, out_ref):
    xs = x_ref[0] * dinv_ref[...]
    out_ref[0, 0] = xs[:, :128]
    out_ref[0, 1] = xs[:, 128:]


def _cell_body(accx_ref, acch_ref, dinv_ref, c_ref, wx_ref, wh_ref, b_ref,
               h_ref, cn_ref, hp_ref):
    d = dinv_ref[...]
    ax = jnp.concatenate([accx_ref[0], accx_ref[1]], axis=1) * d
    ah = jnp.concatenate([acch_ref[0], acch_ref[1]], axis=1) * d
    gates = (jnp.dot(ax, wx_ref[...], preferred_element_type=jnp.float32)
             + jnp.dot(ah, wh_ref[...], preferred_element_type=jnp.float32)
             + b_ref[...])
    i = jax.nn.sigmoid(gates[:, 0 * HH:1 * HH])
    f = jax.nn.sigmoid(gates[:, 1 * HH:2 * HH])
    g = jnp.tanh(gates[:, 2 * HH:3 * HH])
    o = jax.nn.sigmoid(gates[:, 3 * HH:4 * HH])
    cn = f * c_ref[...] + i * g
    h = o * jnp.tanh(cn)
    h_ref[...] = h
    cn_ref[...] = cn
    hp = h * d
    hp_ref[0, 0] = hp[:, :128]
    hp_ref[0, 1] = hp[:, 128:]


def _fc_body(h_ref, w_ref, b_ref, o_ref):
    o_ref[...] = jax.nn.sigmoid(
        jnp.dot(h_ref[...], w_ref[...], preferred_element_type=jnp.float32)
        + b_ref[...])


_BM = 256

_cell_call = pl.pallas_call(
    _cell_body,
    grid=(NP // _BM,),
    in_specs=[
        pl.BlockSpec((2, _BM, 128), lambda n: (0, n, 0)),
        pl.BlockSpec((2, _BM, 128), lambda n: (0, n, 0)),
        pl.BlockSpec((_BM, 1), lambda n: (n, 0)),
        pl.BlockSpec((_BM, HH), lambda n: (n, 0)),
        pl.BlockSpec((DD, 4 * HH), lambda n: (0, 0)),
        pl.BlockSpec((HH, 4 * HH), lambda n: (0, 0)),
        pl.BlockSpec((1, 4 * HH), lambda n: (0, 0)),
    ],
    out_specs=[
        pl.BlockSpec((_BM, HH), lambda n: (n, 0)),
        pl.BlockSpec((_BM, HH), lambda n: (n, 0)),
        pl.BlockSpec((1, 2, _BM, 128), lambda n: (0, 0, n, 0)),
    ],
    out_shape=[
        jax.ShapeDtypeStruct((NP, HH), jnp.float32),
        jax.ShapeDtypeStruct((NP, HH), jnp.float32),
        jax.ShapeDtypeStruct((1, 2, NP, 128), jnp.float32),
    ],
)

_xprep_call = pl.pallas_call(
    _xprep_body,
    grid=(TT, NP // _BM),
    in_specs=[
        pl.BlockSpec((1, _BM, DD), lambda t, n: (t, n, 0)),
        pl.BlockSpec((_BM, 1), lambda t, n: (n, 0)),
    ],
    out_specs=pl.BlockSpec((1, 2, _BM, 128), lambda t, n: (t, 0, n, 0)),
    out_shape=jax.ShapeDtypeStruct((TT, 2, NP, 128), jnp.float32),
)

_fc_call = pl.pallas_call(
    _fc_body,
    out_shape=jax.ShapeDtypeStruct((NP, 128), jnp.float32),
)


def kernel(x, edge_index, Wx0, Wh0, b0, Wx1, Wh1, b1, Wfc, bfc):
    src = edge_index[0].astype(jnp.int32)
    dst = edge_index[1].astype(jnp.int32)

    pad = EP - EE
    src_p = jnp.concatenate([src, jnp.full((pad,), NN, jnp.int32)])
    dst_p = jnp.concatenate([dst, jnp.full((pad,), NN, jnp.int32)])
    srcs = jnp.stack([src_p, src_p + NP]).reshape(NC, NS, NB, BB)
    dsts = dst_p.reshape(NS, NB, BB)

    zeros_agg = jnp.zeros((ZR, 128), jnp.float32)
    ones_tbl = jnp.ones((2 * NP, 128), jnp.float32)

    sc_agg_f = _sc_kernels()
    degp = sc_agg_f(ones_tbl, srcs, dsts, zeros_agg)
    deg = degp[:NP, 0]
    dinv = jax.lax.rsqrt(jnp.clip(deg, 1.0, None)).reshape(NP, 1)

    xpad = jnp.pad(x, ((0, 0), (0, NP - NN), (0, 0)))
    xp = _xprep_call(xpad, dinv).reshape(TT, 2 * NP, 128)

    agg = lambda tbl: sc_agg_f(tbl, srcs, dsts, zeros_agg)

    ax = [agg(xp[t]) for t in range(TT)]

    z2 = jnp.zeros((2, NP, 128), jnp.float32)
    zN = jnp.zeros((NP, HH), jnp.float32)
    b0r = b0.reshape(1, 4 * HH)
    b1r = b1.reshape(1, 4 * HH)

    g0 = z2
    g1 = z2
    c0 = zN
    c1 = zN
    h1 = zN
    for t in range(TT):
        axt = ax[t].reshape(2, NP, 128)
        _, c0, hp0 = _cell_call(axt, g0, dinv, c0, Wx0, Wh0, b0r)
        g0f = agg(hp0.reshape(2 * NP, 128))
        g0 = g0f.reshape(2, NP, 128)
        h1, c1, hp1 = _cell_call(g0, g1, dinv, c1, Wx1, Wh1, b1r)
        if t < TT - 1:
            g1 = agg(hp1.reshape(2 * NP, 128)).reshape(2, NP, 128)

    Wfc_pad = jnp.pad(Wfc, ((0, 0), (0, 127)))
    bfc_pad = jnp.pad(bfc, ((0, 127))).reshape(1, 128)
    score = _fc_call(h1, Wfc_pad, bfc_pad)
    return score[:NN, :1]
